# baseline (speedup 1.0000x reference)
import jax
import jax.numpy as jnp
from jax import lax
from jax.experimental import pallas as pl
from jax.experimental.pallas import tpu as pltpu
from jax.experimental.pallas import tpu_sc as plsc

B = 16384
NN_DIM = 64
MF_DIM = 32
NROWS = 100000
PK = 128

_NC = 2
_NS = 16
_NW = _NC * _NS
_BPW = B // _NW
_ICH = 128
_NICH = _BPW // _ICH

_BLKR = 12800
_BLK = 16384
_NBLK = B // _BLK


def _pack_body(nn, mf, e64, e32, out):
    hp = lax.Precision.HIGHEST
    f32 = jnp.float32
    nnr = lax.dot_general(nn[...], e64[...], (((0,), (0,)), ((), ())),
                          precision=hp, preferred_element_type=f32)
    mfr = lax.dot_general(mf[...], e32[...], (((0,), (0,)), ((), ())),
                          precision=hp, preferred_element_type=f32)
    out[...] = jnp.concatenate(
        [nnr, mfr, jnp.zeros((nnr.shape[0], PK - NN_DIM - MF_DIM), jnp.float32)],
        axis=1)


def _pack(nnT, mfT):
    return pl.pallas_call(
        _pack_body,
        grid=(pl.cdiv(NROWS, _BLKR),),
        in_specs=[
            pl.BlockSpec((NN_DIM, _BLKR), lambda i: (0, i)),
            pl.BlockSpec((MF_DIM, _BLKR), lambda i: (0, i)),
            pl.BlockSpec((NN_DIM, NN_DIM), lambda i: (0, 0)),
            pl.BlockSpec((MF_DIM, MF_DIM), lambda i: (0, 0)),
        ],
        out_specs=pl.BlockSpec((_BLKR, PK), lambda i: (i, 0)),
        out_shape=jax.ShapeDtypeStruct((NROWS, PK), jnp.float32),
        compiler_params=pltpu.CompilerParams(vmem_limit_bytes=100 * 1024 * 1024),
    )(nnT, mfT, jnp.eye(NN_DIM, dtype=jnp.float32), jnp.eye(MF_DIM, dtype=jnp.float32))


def _gather_body(idx_hbm, tab_hbm, out_g, idx_v, dst, sem):
    wid = lax.axis_index("s") * _NC + lax.axis_index("c")
    base = wid * _BPW
    pltpu.sync_copy(idx_hbm.at[pl.ds(base, _BPW)], idx_v)
    copies = []
    for c in range(_NICH):
        sl = pl.ds(c * _ICH, _ICH)
        copies.append(pltpu.async_copy(tab_hbm.at[idx_v.at[sl]], dst.at[sl, :], sem))
    for cp in copies:
        cp.wait()
    pltpu.sync_copy(dst, out_g.at[pl.ds(base, _BPW), :])


def _sc_gather(idx, tab):
    mesh = plsc.VectorSubcoreMesh(core_axis_name="c", subcore_axis_name="s")
    return pl.kernel(
        _gather_body,
        out_type=jax.ShapeDtypeStruct((B, PK), jnp.float32),
        mesh=mesh,
        scratch_types=[
            pltpu.VMEM((_BPW,), jnp.int32),
            pltpu.VMEM((_BPW, PK), jnp.float32),
            pltpu.SemaphoreType.DMA,
        ],
        compiler_params=pltpu.CompilerParams(needs_layout_passes=False),
    )(idx, tab)


def _mlp_body(gu, gi, a1, c1, b1, w2, b2, w3, b3, wmfe, wx, bo, out):
    hp = lax.Precision.DEFAULT
    f32 = jnp.float32
    x = jnp.dot(gu[...], a1[...], precision=hp, preferred_element_type=f32)
    x = x + jnp.dot(gi[...], c1[...], precision=hp, preferred_element_type=f32)
    x = jnp.maximum(x + b1[...], 0.0)
    x = jnp.maximum(jnp.dot(x, w2[...], precision=hp, preferred_element_type=f32) + b2[...], 0.0)
    x = jnp.maximum(jnp.dot(x, w3[...], precision=hp, preferred_element_type=f32) + b3[...], 0.0)
    mf = jnp.dot(gu[...] * gi[...], wmfe[...], precision=hp, preferred_element_type=f32)
    acc = jnp.dot(x, wx[...], precision=hp, preferred_element_type=f32) + mf + bo[0, 0]
    out[...] = acc


def kernel(user, item, mf_usr, mf_item, nn_usr, nn_item,
           fc1_w, fc1_b, fc2_w, fc2_b, fc3_w, fc3_b, neumf_w, neumf_b):
    user = user.astype(jnp.int32)
    item = item.astype(jnp.int32)

    tab_u = _pack(nn_usr.T, mf_usr.T)
    gu = _sc_gather(user, tab_u)
    tab_i = _pack(nn_item.T, mf_item.T)
    gi = _sc_gather(item, tab_i)

    w1 = fc1_w.T
    zpad = jnp.zeros((PK - NN_DIM, 128), jnp.float32)
    a1 = jnp.concatenate([w1[:NN_DIM], zpad], axis=0)
    c1 = jnp.concatenate([w1[NN_DIM:], zpad], axis=0)
    w2 = fc2_w.T
    w3 = fc3_w.T
    wmfe = jnp.concatenate(
        [jnp.zeros((1, NN_DIM), jnp.float32), neumf_w[:, :MF_DIM],
         jnp.zeros((1, PK - NN_DIM - MF_DIM), jnp.float32)], axis=1).T
    wx = neumf_w[:, MF_DIM:].T

    full = lambda shape: pl.BlockSpec(shape, lambda i: (0, 0))
    out2d = pl.pallas_call(
        _mlp_body,
        grid=(_NBLK,),
        in_specs=[
            pl.BlockSpec((_BLK, PK), lambda i: (i, 0)),
            pl.BlockSpec((_BLK, PK), lambda i: (i, 0)),
            full((PK, 128)), full((PK, 128)), full((1, 128)),
            full((128, 64)), full((1, 64)),
            full((64, 32)), full((1, 32)),
            full((PK, 1)), full((32, 1)), full((1, 1)),
        ],
        out_specs=pl.BlockSpec((_BLK, 1), lambda i: (i, 0)),
        out_shape=jax.ShapeDtypeStruct((B, 1), jnp.float32),
    )(gu, gi, a1, c1, fc1_b[None], w2, fc2_b[None],
      w3, fc3_b[None], wmfe, wx, neumf_b[None])
    return out2d.reshape(B)


# speedup vs baseline: 1.7991x; 1.7991x over previous
import jax
import jax.numpy as jnp
from jax import lax
from jax.experimental import pallas as pl
from jax.experimental.pallas import tpu as pltpu
from jax.experimental.pallas import tpu_sc as plsc

B = 16384
NN_DIM = 64
MF_DIM = 32
NROWS = 100000
PK = 128

_NC = 2
_NS = 16
_NW = _NC * _NS
_BPW = B // _NW
_ICH = 128
_NICH = _BPW // _ICH

_BLKR = 12800
_BLK = 16384
_NBLK = B // _BLK


def _pack_body(nn, mf, e64, e32, out):
    hp = lax.Precision.DEFAULT
    f32 = jnp.float32
    nnr = lax.dot_general(nn[...], e64[...], (((0,), (0,)), ((), ())),
                          precision=hp, preferred_element_type=f32)
    mfr = lax.dot_general(mf[...], e32[...], (((0,), (0,)), ((), ())),
                          precision=hp, preferred_element_type=f32)
    out[...] = jnp.concatenate(
        [nnr, mfr, jnp.zeros((nnr.shape[0], PK - NN_DIM - MF_DIM), jnp.float32)],
        axis=1)


def _pack(nnT, mfT):
    return pl.pallas_call(
        _pack_body,
        grid=(pl.cdiv(NROWS, _BLKR),),
        in_specs=[
            pl.BlockSpec((NN_DIM, _BLKR), lambda i: (0, i)),
            pl.BlockSpec((MF_DIM, _BLKR), lambda i: (0, i)),
            pl.BlockSpec((NN_DIM, NN_DIM), lambda i: (0, 0)),
            pl.BlockSpec((MF_DIM, MF_DIM), lambda i: (0, 0)),
        ],
        out_specs=pl.BlockSpec((_BLKR, PK), lambda i: (i, 0)),
        out_shape=jax.ShapeDtypeStruct((NROWS, PK), jnp.float32),
        compiler_params=pltpu.CompilerParams(vmem_limit_bytes=100 * 1024 * 1024),
    )(nnT, mfT, jnp.eye(NN_DIM, dtype=jnp.float32), jnp.eye(MF_DIM, dtype=jnp.float32))


def _gather_body(idx_hbm, tab_hbm, out_g, idx_v, dst, sem):
    wid = lax.axis_index("s") * _NC + lax.axis_index("c")
    base = wid * _BPW
    pltpu.sync_copy(idx_hbm.at[pl.ds(base, _BPW)], idx_v)
    copies = []
    for c in range(_NICH):
        sl = pl.ds(c * _ICH, _ICH)
        copies.append(pltpu.async_copy(tab_hbm.at[idx_v.at[sl]], dst.at[sl, :], sem))
    for cp in copies:
        cp.wait()
    pltpu.sync_copy(dst, out_g.at[pl.ds(base, _BPW), :])


def _sc_gather(idx, tab):
    mesh = plsc.VectorSubcoreMesh(core_axis_name="c", subcore_axis_name="s")
    return pl.kernel(
        _gather_body,
        out_type=jax.ShapeDtypeStruct((B, PK), jnp.float32),
        mesh=mesh,
        scratch_types=[
            pltpu.VMEM((_BPW,), jnp.int32),
            pltpu.VMEM((_BPW, PK), jnp.float32),
            pltpu.SemaphoreType.DMA,
        ],
        compiler_params=pltpu.CompilerParams(needs_layout_passes=False),
    )(idx, tab)


def _mlp_body(gu, gi, a1, c1, b1, w2, b2, w3, b3, wmfe, wx, bo, out):
    hp = lax.Precision.DEFAULT
    f32 = jnp.float32
    x = jnp.dot(gu[...], a1[...], precision=hp, preferred_element_type=f32)
    x = x + jnp.dot(gi[...], c1[...], precision=hp, preferred_element_type=f32)
    x = jnp.maximum(x + b1[...], 0.0)
    x = jnp.maximum(jnp.dot(x, w2[...], precision=hp, preferred_element_type=f32) + b2[...], 0.0)
    x = jnp.maximum(jnp.dot(x, w3[...], precision=hp, preferred_element_type=f32) + b3[...], 0.0)
    mf = jnp.dot(gu[...] * gi[...], wmfe[...], precision=hp, preferred_element_type=f32)
    acc = jnp.dot(x, wx[...], precision=hp, preferred_element_type=f32) + mf + bo[0, 0]
    out[...] = acc


def kernel(user, item, mf_usr, mf_item, nn_usr, nn_item,
           fc1_w, fc1_b, fc2_w, fc2_b, fc3_w, fc3_b, neumf_w, neumf_b):
    user = user.astype(jnp.int32)
    item = item.astype(jnp.int32)

    tab_u = _pack(nn_usr.T, mf_usr.T)
    gu = _sc_gather(user, tab_u)
    tab_i = _pack(nn_item.T, mf_item.T)
    gi = _sc_gather(item, tab_i)

    w1 = fc1_w.T
    zpad = jnp.zeros((PK - NN_DIM, 128), jnp.float32)
    a1 = jnp.concatenate([w1[:NN_DIM], zpad], axis=0)
    c1 = jnp.concatenate([w1[NN_DIM:], zpad], axis=0)
    w2 = fc2_w.T
    w3 = fc3_w.T
    wmfe = jnp.concatenate(
        [jnp.zeros((1, NN_DIM), jnp.float32), neumf_w[:, :MF_DIM],
         jnp.zeros((1, PK - NN_DIM - MF_DIM), jnp.float32)], axis=1).T
    wx = neumf_w[:, MF_DIM:].T

    full = lambda shape: pl.BlockSpec(shape, lambda i: (0, 0))
    out2d = pl.pallas_call(
        _mlp_body,
        grid=(_NBLK,),
        in_specs=[
            pl.BlockSpec((_BLK, PK), lambda i: (i, 0)),
            pl.BlockSpec((_BLK, PK), lambda i: (i, 0)),
            full((PK, 128)), full((PK, 128)), full((1, 128)),
            full((128, 64)), full((1, 64)),
            full((64, 32)), full((1, 32)),
            full((PK, 1)), full((32, 1)), full((1, 1)),
        ],
        out_specs=pl.BlockSpec((_BLK, 1), lambda i: (i, 0)),
        out_shape=jax.ShapeDtypeStruct((B, 1), jnp.float32),
    )(gu, gi, a1, c1, fc1_b[None], w2, fc2_b[None],
      w3, fc3_b[None], wmfe, wx, neumf_b[None])
    return out2d.reshape(B)


# speedup vs baseline: 1.8134x; 1.0079x over previous
import jax
import jax.numpy as jnp
from jax import lax
from jax.experimental import pallas as pl
from jax.experimental.pallas import tpu as pltpu
from jax.experimental.pallas import tpu_sc as plsc

B = 16384
NN_DIM = 64
MF_DIM = 32
NROWS = 100000
PK = 128

_NC = 2
_NS = 16
_NW = _NC * _NS
_BPW = B // _NW
_ICH = 128
_NICH = _BPW // _ICH

_BLKR = 12800
_BLK = 16384
_NBLK = B // _BLK


def _pack_body(nn, mf, out):
    nnr = nn[...].T
    mfr = mf[...].T
    out[...] = jnp.concatenate(
        [nnr, mfr, jnp.zeros((nnr.shape[0], PK - NN_DIM - MF_DIM), jnp.float32)],
        axis=1)


def _pack(nnT, mfT):
    return pl.pallas_call(
        _pack_body,
        grid=(pl.cdiv(NROWS, _BLKR),),
        in_specs=[
            pl.BlockSpec((NN_DIM, _BLKR), lambda i: (0, i)),
            pl.BlockSpec((MF_DIM, _BLKR), lambda i: (0, i)),
        ],
        out_specs=pl.BlockSpec((_BLKR, PK), lambda i: (i, 0)),
        out_shape=jax.ShapeDtypeStruct((NROWS, PK), jnp.float32),
        compiler_params=pltpu.CompilerParams(vmem_limit_bytes=100 * 1024 * 1024),
    )(nnT, mfT)


def _gather_body(idx_hbm, tab_hbm, out_g, idx_v, dst, sem):
    wid = lax.axis_index("s") * _NC + lax.axis_index("c")
    base = wid * _BPW
    pltpu.sync_copy(idx_hbm.at[pl.ds(base, _BPW)], idx_v)
    copies = []
    for c in range(_NICH):
        sl = pl.ds(c * _ICH, _ICH)
        copies.append(pltpu.async_copy(tab_hbm.at[idx_v.at[sl]], dst.at[sl, :], sem))
    for cp in copies:
        cp.wait()
    pltpu.sync_copy(dst, out_g.at[pl.ds(base, _BPW), :])


def _sc_gather(idx, tab):
    mesh = plsc.VectorSubcoreMesh(core_axis_name="c", subcore_axis_name="s")
    return pl.kernel(
        _gather_body,
        out_type=jax.ShapeDtypeStruct((B, PK), jnp.float32),
        mesh=mesh,
        scratch_types=[
            pltpu.VMEM((_BPW,), jnp.int32),
            pltpu.VMEM((_BPW, PK), jnp.float32),
            pltpu.SemaphoreType.DMA,
        ],
        compiler_params=pltpu.CompilerParams(needs_layout_passes=False),
    )(idx, tab)


def _mlp_body(gu, gi, a1, c1, b1, w2, b2, w3, b3, wmfe, wx, bo, out):
    hp = lax.Precision.DEFAULT
    f32 = jnp.float32
    x = jnp.dot(gu[...], a1[...], precision=hp, preferred_element_type=f32)
    x = x + jnp.dot(gi[...], c1[...], precision=hp, preferred_element_type=f32)
    x = jnp.maximum(x + b1[...], 0.0)
    x = jnp.maximum(jnp.dot(x, w2[...], precision=hp, preferred_element_type=f32) + b2[...], 0.0)
    x = jnp.maximum(jnp.dot(x, w3[...], precision=hp, preferred_element_type=f32) + b3[...], 0.0)
    mf = jnp.dot(gu[...] * gi[...], wmfe[...], precision=hp, preferred_element_type=f32)
    acc = jnp.dot(x, wx[...], precision=hp, preferred_element_type=f32) + mf + bo[0, 0]
    out[...] = acc


def kernel(user, item, mf_usr, mf_item, nn_usr, nn_item,
           fc1_w, fc1_b, fc2_w, fc2_b, fc3_w, fc3_b, neumf_w, neumf_b):
    user = user.astype(jnp.int32)
    item = item.astype(jnp.int32)

    tab_u = _pack(nn_usr.T, mf_usr.T)
    gu = _sc_gather(user, tab_u)
    tab_i = _pack(nn_item.T, mf_item.T)
    gi = _sc_gather(item, tab_i)

    w1 = fc1_w.T
    zpad = jnp.zeros((PK - NN_DIM, 128), jnp.float32)
    a1 = jnp.concatenate([w1[:NN_DIM], zpad], axis=0)
    c1 = jnp.concatenate([w1[NN_DIM:], zpad], axis=0)
    w2 = fc2_w.T
    w3 = fc3_w.T
    wmfe = jnp.concatenate(
        [jnp.zeros((1, NN_DIM), jnp.float32), neumf_w[:, :MF_DIM],
         jnp.zeros((1, PK - NN_DIM - MF_DIM), jnp.float32)], axis=1).T
    wx = neumf_w[:, MF_DIM:].T

    full = lambda shape: pl.BlockSpec(shape, lambda i: (0, 0))
    out2d = pl.pallas_call(
        _mlp_body,
        grid=(_NBLK,),
        in_specs=[
            pl.BlockSpec((_BLK, PK), lambda i: (i, 0)),
            pl.BlockSpec((_BLK, PK), lambda i: (i, 0)),
            full((PK, 128)), full((PK, 128)), full((1, 128)),
            full((128, 64)), full((1, 64)),
            full((64, 32)), full((1, 32)),
            full((PK, 1)), full((32, 1)), full((1, 1)),
        ],
        out_specs=pl.BlockSpec((_BLK, 1), lambda i: (i, 0)),
        out_shape=jax.ShapeDtypeStruct((B, 1), jnp.float32),
    )(gu, gi, a1, c1, fc1_b[None], w2, fc2_b[None],
      w3, fc3_b[None], wmfe, wx, neumf_b[None])
    return out2d.reshape(B)


# speedup vs baseline: 1.8449x; 1.0174x over previous
import jax
import jax.numpy as jnp
from jax import lax
from jax.experimental import pallas as pl
from jax.experimental.pallas import tpu as pltpu
from jax.experimental.pallas import tpu_sc as plsc

B = 16384
NN_DIM = 64
MF_DIM = 32
NROWS = 100000
PK = 128

_NC = 2
_NS = 16
_NW = _NC * _NS
_BPW = B // _NW
_ICH = 128
_NICH = _BPW // _ICH

_BLKR = 12800
_BLK = 16384
_NBLK = B // _BLK


def _pack_body(nn, mf, out):
    nnr = nn[...].T
    mfr = mf[...].T
    out[...] = jnp.concatenate(
        [nnr, mfr, jnp.zeros((nnr.shape[0], PK - NN_DIM - MF_DIM), jnp.float32)],
        axis=1)


def _pack(nnT, mfT):
    return pl.pallas_call(
        _pack_body,
        grid=(pl.cdiv(NROWS, _BLKR),),
        in_specs=[
            pl.BlockSpec((NN_DIM, _BLKR), lambda i: (0, i)),
            pl.BlockSpec((MF_DIM, _BLKR), lambda i: (0, i)),
        ],
        out_specs=pl.BlockSpec((_BLKR, PK), lambda i: (i, 0)),
        out_shape=jax.ShapeDtypeStruct((NROWS, PK), jnp.float32),
        compiler_params=pltpu.CompilerParams(vmem_limit_bytes=100 * 1024 * 1024),
    )(nnT, mfT)


def _gather_body(idx_hbm, tab_hbm, out_g, idx_v, dst, sem):
    wid = lax.axis_index("s") * _NC + lax.axis_index("c")
    base = wid * _BPW
    pltpu.sync_copy(idx_hbm.at[pl.ds(base, _BPW)], idx_v)
    copies = []
    for c in range(_NICH):
        sl = pl.ds(c * _ICH, _ICH)
        copies.append(pltpu.async_copy(tab_hbm.at[idx_v.at[sl]], dst.at[sl, :], sem))
    for cp in copies:
        cp.wait()
    pltpu.sync_copy(dst, out_g.at[pl.ds(base, _BPW), :])


def _sc_gather(idx, tab):
    mesh = plsc.VectorSubcoreMesh(core_axis_name="c", subcore_axis_name="s")
    return pl.kernel(
        _gather_body,
        out_type=jax.ShapeDtypeStruct((B, PK), jnp.float32),
        mesh=mesh,
        scratch_types=[
            pltpu.VMEM((_BPW,), jnp.int32),
            pltpu.VMEM((_BPW, PK), jnp.float32),
            pltpu.SemaphoreType.DMA,
        ],
        compiler_params=pltpu.CompilerParams(needs_layout_passes=False),
    )(idx, tab)


def _mlp_body(gu, gi, a1, c1, b1, w2, b2, w3, b3, wmfe, wx, bo, out):
    hp = lax.Precision.DEFAULT
    f32 = jnp.float32
    x = jnp.dot(gu[...], a1[...], precision=hp, preferred_element_type=f32)
    x = x + jnp.dot(gi[...], c1[...], precision=hp, preferred_element_type=f32)
    x = jnp.maximum(x + b1[...], 0.0)
    x = jnp.maximum(jnp.dot(x, w2[...], precision=hp, preferred_element_type=f32) + b2[...], 0.0)
    x = jnp.maximum(jnp.dot(x, w3[...], precision=hp, preferred_element_type=f32) + b3[...], 0.0)
    mf = jnp.dot(gu[...] * gi[...], wmfe[...], precision=hp, preferred_element_type=f32)
    acc = jnp.dot(x, wx[...], precision=hp, preferred_element_type=f32) + mf + bo[0, 0]
    out[...] = acc.reshape(-1)


def kernel(user, item, mf_usr, mf_item, nn_usr, nn_item,
           fc1_w, fc1_b, fc2_w, fc2_b, fc3_w, fc3_b, neumf_w, neumf_b):
    user = user.astype(jnp.int32)
    item = item.astype(jnp.int32)

    tab_u = _pack(nn_usr.T, mf_usr.T)
    gu = _sc_gather(user, tab_u)
    tab_i = _pack(nn_item.T, mf_item.T)
    gi = _sc_gather(item, tab_i)

    w1 = fc1_w.T
    zpad = jnp.zeros((PK - NN_DIM, 128), jnp.float32)
    a1 = jnp.concatenate([w1[:NN_DIM], zpad], axis=0)
    c1 = jnp.concatenate([w1[NN_DIM:], zpad], axis=0)
    w2 = fc2_w.T
    w3 = fc3_w.T
    wmfe = jnp.concatenate(
        [jnp.zeros((1, NN_DIM), jnp.float32), neumf_w[:, :MF_DIM],
         jnp.zeros((1, PK - NN_DIM - MF_DIM), jnp.float32)], axis=1).T
    wx = neumf_w[:, MF_DIM:].T

    full = lambda shape: pl.BlockSpec(shape, lambda i: (0, 0))
    out2d = pl.pallas_call(
        _mlp_body,
        grid=(_NBLK,),
        in_specs=[
            pl.BlockSpec((_BLK, PK), lambda i: (i, 0)),
            pl.BlockSpec((_BLK, PK), lambda i: (i, 0)),
            full((PK, 128)), full((PK, 128)), full((1, 128)),
            full((128, 64)), full((1, 64)),
            full((64, 32)), full((1, 32)),
            full((PK, 1)), full((32, 1)), full((1, 1)),
        ],
        out_specs=pl.BlockSpec((_BLK,), lambda i: (i,)),
        out_shape=jax.ShapeDtypeStruct((B,), jnp.float32),
    )(gu, gi, a1, c1, fc1_b[None], w2, fc2_b[None],
      w3, fc3_b[None], wmfe, wx, neumf_b[None])
    return out2d
